# Initial kernel scaffold; baseline (speedup 1.0000x reference)
import jax
import jax.numpy as jnp
from jax.experimental import pallas as pl


def kernel(X, edge_features, edge_index, params):
    raise NotImplementedError("write your pallas kernel here")


# speedup vs baseline: 4.2205x; 4.2205x over previous
import functools

import jax
import jax.numpy as jnp
from jax import lax
from jax.experimental import pallas as pl
from jax.experimental.pallas import tpu as pltpu
from jax.experimental.pallas import tpu_sc as plsc

NEG_SLOPE = 0.01
NC = 2
NS = 16
CH = 128

BE = 4000
BN = 2000


def _leaky(x):
    return jnp.where(x >= 0, x, NEG_SLOPE * x)


def _sig(x):
    return jax.nn.sigmoid(x)


def _full_spec(shape):
    return pl.BlockSpec(shape, lambda i: tuple(0 for _ in shape))


def _row_spec(rows, cols, off=0):
    return pl.BlockSpec((rows, cols), lambda i, off=off: (i + off, 0))


def _edge_mlp(efr, esc, We1, be1, We2, be2, Wad1, bad1, Wad2, bad2,
              War1, bar1, War2, bar2):
    E_, F = efr.shape

    def body(efr_ref, esc_ref, We1_ref, be1_ref, We2_ref, be2_ref,
             Wad1_ref, bad1_ref, Wad2_ref, bad2_ref,
             War1_ref, bar1_ref, War2_ref, bar2_ref,
             efnn_ref, wd_ref, wr_ref):
        ef = (efr_ref[...] + 0.0001) / esc_ref[...]
        h = _leaky(jnp.dot(ef, We1_ref[...], preferred_element_type=jnp.float32)
                   + be1_ref[...])
        efnn = _leaky(jnp.dot(h, We2_ref[...], preferred_element_type=jnp.float32)
                      + be2_ref[...])
        efnn_ref[...] = efnn
        ad = _leaky(jnp.dot(efnn, Wad1_ref[...], preferred_element_type=jnp.float32)
                    + bad1_ref[...])
        alpha_d = _sig(jnp.dot(ad, Wad2_ref[...], preferred_element_type=jnp.float32)
                       + bad2_ref[...])
        ar = _leaky(jnp.dot(efnn, War1_ref[...], preferred_element_type=jnp.float32)
                    + bar1_ref[...])
        alpha_r = _sig(jnp.dot(ar, War2_ref[...], preferred_element_type=jnp.float32)
                       + bar2_ref[...])
        onehot = (lax.broadcasted_iota(jnp.int32, (1, 16), 1) == 0).astype(jnp.float32)
        wd_ref[...] = alpha_d * onehot
        wr_ref[...] = alpha_r * onehot

    g = E_ // BE
    return pl.pallas_call(
        body,
        grid=(g,),
        in_specs=[
            _row_spec(BE, F),
            _full_spec(esc.shape), _full_spec(We1.shape), _full_spec(be1.shape),
            _full_spec(We2.shape), _full_spec(be2.shape),
            _full_spec(Wad1.shape), _full_spec(bad1.shape),
            _full_spec(Wad2.shape), _full_spec(bad2.shape),
            _full_spec(War1.shape), _full_spec(bar1.shape),
            _full_spec(War2.shape), _full_spec(bar2.shape),
        ],
        out_specs=[_row_spec(BE, 16), _row_spec(BE, 16), _row_spec(BE, 16)],
        out_shape=[jax.ShapeDtypeStruct((E_, 16), jnp.float32)] * 3,
    )(efr, esc, We1, be1, We2, be2, Wad1, bad1, Wad2, bad2,
      War1, bar1, War2, bar2)


def _node_mlp(X, nsc, Wn1, bn1, Wn2, bn2):
    N_, F = X.shape

    def body(X_ref, nsc_ref, W1_ref, b1_ref, W2_ref, b2_ref, out_ref):
        Xs = X_ref[...] / nsc_ref[...]
        h = _leaky(jnp.dot(Xs, W1_ref[...], preferred_element_type=jnp.float32)
                   + b1_ref[...])
        out_ref[...] = _leaky(jnp.dot(h, W2_ref[...], preferred_element_type=jnp.float32)
                              + b2_ref[...])

    g = N_ // BN
    return pl.pallas_call(
        body,
        grid=(g,),
        in_specs=[_row_spec(BN, F), _full_spec(nsc.shape),
                  _full_spec(Wn1.shape), _full_spec(bn1.shape),
                  _full_spec(Wn2.shape), _full_spec(bn2.shape)],
        out_specs=_row_spec(BN, 16),
        out_shape=jax.ShapeDtypeStruct((N_, 16), jnp.float32),
    )(X, nsc, Wn1, bn1, Wn2, bn2)


def _msg_mlp(g2, wd, wr, Wat, Wab, ba, Wb, bb):
    T, D = g2.shape
    E_ = T // 2
    g = E_ // BE

    def body(xs_ref, xd_ref, wd_ref, wr_ref, Wat_ref, Wab_ref, ba_ref,
             Wb_ref, bb_ref, md_ref, mr_ref):
        x1 = xs_ref[...]
        x2 = xd_ref[...]
        d = x2 - x1
        h = _leaky(jnp.dot(x1, Wat_ref[...], preferred_element_type=jnp.float32)
                   + jnp.dot(d, Wab_ref[...], preferred_element_type=jnp.float32)
                   + ba_ref[...])
        m = _leaky(jnp.dot(h, Wb_ref[...], preferred_element_type=jnp.float32)
                   + bb_ref[...])
        md_ref[...] = m * wd_ref[..., 0:1]
        h2 = _leaky(jnp.dot(x2, Wat_ref[...], preferred_element_type=jnp.float32)
                    - jnp.dot(d, Wab_ref[...], preferred_element_type=jnp.float32)
                    + ba_ref[...])
        m2 = _leaky(jnp.dot(h2, Wb_ref[...], preferred_element_type=jnp.float32)
                    + bb_ref[...])
        mr_ref[...] = m2 * wr_ref[..., 0:1]

    return pl.pallas_call(
        body,
        grid=(g,),
        in_specs=[
            _row_spec(BE, D),
            pl.BlockSpec((BE, D), lambda i, _g=g: (i + _g, 0)),
            _row_spec(BE, 16), _row_spec(BE, 16),
            _full_spec(Wat.shape), _full_spec(Wab.shape), _full_spec(ba.shape),
            _full_spec(Wb.shape), _full_spec(bb.shape),
        ],
        out_specs=[_row_spec(BE, 16), _row_spec(BE, 16)],
        out_shape=[jax.ShapeDtypeStruct((E_, 16), jnp.float32)] * 2,
    )(g2, g2, wd, wr, Wat, Wab, ba, Wb, bb)


def _normalize(macc, wacc, e0, Wat, ba, Wb, bb, nfloat):
    TwoN, D = macc.shape
    N_ = TwoN // 2
    g = N_ // BN

    def body(p0_ref, p1_ref, w0_ref, w1_ref, e0_ref, Wat_ref, ba_ref,
             Wb_ref, bb_ref, out_ref):
        agg = p0_ref[...] + p1_ref[...]
        ws = w0_ref[..., 0:1] + w1_ref[..., 0:1]
        h0 = _leaky(jnp.dot(e0_ref[...], Wat_ref[...],
                            preferred_element_type=jnp.float32) + ba_ref[...])
        m0 = _leaky(jnp.dot(h0, Wb_ref[...], preferred_element_type=jnp.float32)
                    + bb_ref[...])
        is0 = ((lax.broadcasted_iota(jnp.int32, (BN, 1), 0) == 0)
               & (pl.program_id(0) == 0)).astype(jnp.float32)
        out_ref[...] = (agg + is0 * (nfloat * m0)) / (ws + is0 * nfloat + 1e-9)

    return pl.pallas_call(
        body,
        grid=(g,),
        in_specs=[
            _row_spec(BN, D),
            pl.BlockSpec((BN, D), lambda i, _g=g: (i + _g, 0)),
            _row_spec(BN, D),
            pl.BlockSpec((BN, D), lambda i, _g=g: (i + _g, 0)),
            _full_spec(e0.shape),
            _full_spec(Wat.shape), _full_spec(ba.shape),
            _full_spec(Wb.shape), _full_spec(bb.shape),
        ],
        out_specs=_row_spec(BN, D),
        out_shape=jax.ShapeDtypeStruct((N_, D), jnp.float32),
    )(macc, macc, wacc, wacc, e0, Wat, ba, Wb, bb)


def _final_mlp(g2, efnn, efr, esc, Ws, Wd, Wen, Wef, ben1, Wen2, ben2,
               Wo1, bo1, Wo2, bo2):
    T, D = g2.shape
    E_ = T // 2
    g = E_ // BE

    def body(gs_ref, gd_ref, efnn_ref, efr_ref, esc_ref,
             Ws_ref, Wd_ref, Wen_ref, Wef_ref, ben1_ref,
             Wen2_ref, ben2_ref, Wo1_ref, bo1_ref, Wo2_ref, bo2_ref,
             out_ref):
        ef = (efr_ref[...] + 0.0001) / esc_ref[...]
        h = _leaky(jnp.dot(gs_ref[...], Ws_ref[...], preferred_element_type=jnp.float32)
                   + jnp.dot(gd_ref[...], Wd_ref[...], preferred_element_type=jnp.float32)
                   + jnp.dot(efnn_ref[...], Wen_ref[...], preferred_element_type=jnp.float32)
                   + jnp.dot(ef, Wef_ref[...], preferred_element_type=jnp.float32)
                   + ben1_ref[...])
        emb = _leaky(jnp.dot(h, Wen2_ref[...], preferred_element_type=jnp.float32)
                     + ben2_ref[...])
        o = _leaky(jnp.dot(emb, Wo1_ref[...], preferred_element_type=jnp.float32)
                   + bo1_ref[...])
        out_ref[...] = _sig(jnp.dot(o, Wo2_ref[...], preferred_element_type=jnp.float32)
                            + bo2_ref[...])

    return pl.pallas_call(
        body,
        grid=(g,),
        in_specs=[
            _row_spec(BE, D),
            pl.BlockSpec((BE, D), lambda i, _g=g: (i + _g, 0)),
            _row_spec(BE, 16), _row_spec(BE, efr.shape[1]),
            _full_spec(esc.shape),
            _full_spec(Ws.shape), _full_spec(Wd.shape), _full_spec(Wen.shape),
            _full_spec(Wef.shape), _full_spec(ben1.shape),
            _full_spec(Wen2.shape), _full_spec(ben2.shape),
            _full_spec(Wo1.shape), _full_spec(bo1.shape),
            _full_spec(Wo2.shape), _full_spec(bo2.shape),
        ],
        out_specs=_row_spec(BE, 1),
        out_shape=jax.ShapeDtypeStruct((E_, 1), jnp.float32),
    )(g2, g2, efnn, efr, esc, Ws, Wd, Wen, Wef, ben1, Wen2, ben2,
      Wo1, bo1, Wo2, bo2)


def _sc_gather(table, src, dst):
    N_, D = table.shape
    E_ = src.shape[0]
    per_w = E_ // NS
    nfull, rem = divmod(per_w, CH)
    mesh = plsc.VectorSubcoreMesh(core_axis_name="c", subcore_axis_name="s",
                                  num_cores=NC)

    @functools.partial(
        pl.kernel, mesh=mesh,
        out_type=jax.ShapeDtypeStruct((2 * E_, D), jnp.float32),
        compiler_params=pltpu.CompilerParams(use_tc_tiling_on_sc=False),
        scratch_types=[
            pltpu.VMEM((CH,), jnp.int32),
            pltpu.VMEM((CH, D), jnp.float32),
            pltpu.VMEM((max(rem, 8),), jnp.int32),
            pltpu.VMEM((max(rem, 8), D), jnp.float32),
        ],
    )
    def k(table_hbm, src_hbm, dst_hbm, out_hbm, idx_v, rows_v, idx_r, rows_r):
        cid = lax.axis_index("c")
        sid = lax.axis_index("s")
        base = sid * per_w
        obase = cid * E_ + base

        def run(ind_hbm):
            @pl.loop(0, nfull)
            def _(t):
                off = base + t * CH
                pltpu.sync_copy(ind_hbm.at[pl.ds(off, CH)], idx_v)
                pltpu.sync_copy(table_hbm.at[idx_v], rows_v)
                pltpu.sync_copy(rows_v, out_hbm.at[pl.ds(obase + t * CH, CH)])
            if rem:
                off = base + nfull * CH
                pltpu.sync_copy(ind_hbm.at[pl.ds(off, rem)],
                                idx_r.at[pl.ds(0, rem)])
                pltpu.sync_copy(table_hbm.at[idx_r.at[pl.ds(0, rem)]],
                                rows_r.at[pl.ds(0, rem)])
                pltpu.sync_copy(rows_r.at[pl.ds(0, rem)],
                                out_hbm.at[pl.ds(obase + nfull * CH, rem)])

        @pl.when(cid == 0)
        def _():
            run(src_hbm)

        @pl.when(cid == 1)
        def _():
            run(dst_hbm)

    return k(table, src, dst)


def _sc_scatter(vd, vr, src, dst, n_out):
    E_, D = vd.shape
    per_w = E_ // NS
    nfull, rem = divmod(per_w, CH)
    rps = n_out // NS
    ZR = 125
    mesh = plsc.VectorSubcoreMesh(core_axis_name="c", subcore_axis_name="s",
                                  num_cores=NC)

    @functools.partial(
        pl.kernel, mesh=mesh,
        out_type=jax.ShapeDtypeStruct((NC * n_out, D), jnp.float32),
        compiler_params=pltpu.CompilerParams(use_tc_tiling_on_sc=False),
        scratch_types=[
            pltpu.VMEM_SHARED((n_out, D), jnp.float32),
            pltpu.VMEM((CH,), jnp.int32),
            pltpu.VMEM((CH, D), jnp.float32),
            pltpu.VMEM((max(rem, 8),), jnp.int32),
            pltpu.VMEM((max(rem, 8), D), jnp.float32),
            pltpu.VMEM((ZR, D), jnp.float32),
        ],
    )
    def k(vd_hbm, vr_hbm, src_hbm, dst_hbm, out_hbm,
          acc, idx_v, val_v, idx_r, val_r, zbuf):
        cid = lax.axis_index("c")
        sid = lax.axis_index("s")

        @pl.loop(0, ZR)
        def _(i):
            zbuf[i, :] = jnp.zeros((D,), jnp.float32)

        @pl.loop(0, rps // ZR)
        def _(j):
            pltpu.sync_copy(zbuf, acc.at[pl.ds(sid * rps + j * ZR, ZR)])

        plsc.subcore_barrier()
        base = sid * per_w

        def run(val_hbm, ind_hbm):
            @pl.loop(0, nfull)
            def _(t):
                off = base + t * CH
                pltpu.sync_copy(ind_hbm.at[pl.ds(off, CH)], idx_v)
                pltpu.sync_copy(val_hbm.at[pl.ds(off, CH)], val_v)
                pltpu.sync_copy(val_v, acc.at[idx_v], add=True)
            if rem:
                off = base + nfull * CH
                pltpu.sync_copy(ind_hbm.at[pl.ds(off, rem)],
                                idx_r.at[pl.ds(0, rem)])
                pltpu.sync_copy(val_hbm.at[pl.ds(off, rem)],
                                val_r.at[pl.ds(0, rem)])
                pltpu.sync_copy(val_r.at[pl.ds(0, rem)],
                                acc.at[idx_r.at[pl.ds(0, rem)]], add=True)

        @pl.when(cid == 0)
        def _():
            run(vd_hbm, src_hbm)

        @pl.when(cid == 1)
        def _():
            run(vr_hbm, dst_hbm)

        plsc.subcore_barrier()
        pltpu.sync_copy(acc.at[pl.ds(sid * rps, rps)],
                        out_hbm.at[pl.ds(cid * n_out + sid * rps, rps)])

    return k(vd, vr, src, dst)


def kernel(X, edge_features, edge_index, params):
    p = params
    N_ = X.shape[0]
    E_ = edge_features.shape[0]
    src = edge_index[:, 0]
    dst = edge_index[:, 1]

    esc = p['edge_scaler'].reshape(1, -1)
    nsc = p['node_scaler'].reshape(1, -1)

    def b(v):
        return v.reshape(1, -1)

    efnn, wd, wr = _edge_mlp(
        edge_features, esc,
        p['We1'], b(p['be1']), p['We2'], b(p['be2']),
        p['Wad1'], b(p['bad1']), p['Wad2'], b(p['bad2']),
        p['War1'], b(p['bar1']), p['War2'], b(p['bar2']))

    emb = _node_mlp(X, nsc, p['Wn1'], b(p['bn1']), p['Wn2'], b(p['bn2']))

    wacc = _sc_scatter(wd, wr, src, dst, N_)

    for i in range(2):
        Wg_a = p['Wg%da' % i]
        Wat, Wab = Wg_a[:16], Wg_a[16:]
        ba = b(p['bg%da' % i])
        Wb = p['Wg%db' % i]
        bb = b(p['bg%db' % i])
        g2 = _sc_gather(emb, src, dst)
        md, mr = _msg_mlp(g2, wd, wr, Wat, Wab, ba, Wb, bb)
        macc = _sc_scatter(md, mr, src, dst, N_)
        e0 = lax.slice(emb, (0, 0), (1, emb.shape[1]))
        emb = _normalize(macc, wacc, e0, Wat, ba, Wb, bb, float(N_))

    g2 = _sc_gather(emb, src, dst)
    Wen1 = p['Wen1']
    pred = _final_mlp(
        g2, efnn, edge_features, esc,
        Wen1[0:16], Wen1[16:32], Wen1[32:48], Wen1[48:60],
        b(p['ben1']), p['Wen2'], b(p['ben2']),
        p['Wo1'], b(p['bo1']), p['Wo2'], b(p['bo2']))
    return pred
